# baseline (speedup 1.0000x reference)
import functools
import math

import jax
import jax.numpy as jnp
from jax import lax
from jax.experimental import pallas as pl
from jax.experimental.pallas import tpu as pltpu
from jax.experimental.pallas import tpu_sc as plsc

EMB_DIM = 64
SCALE = math.sqrt(EMB_DIM)

NUM_CORES = 2
NUM_SUBCORES = 16
NUM_WORKERS = NUM_CORES * NUM_SUBCORES
CHUNK = 128
LANES = 16
NBUF = 4


TBLK = 4096


def _transpose_scale(wt):
    v = wt.shape[1]

    def tbody(in_ref, out_ref):
        out_ref[...] = in_ref[...].T * SCALE

    return pl.pallas_call(
        tbody,
        grid=(pl.cdiv(v, TBLK),),
        in_specs=[pl.BlockSpec((EMB_DIM, TBLK), lambda i: (0, i))],
        out_specs=pl.BlockSpec((TBLK, EMB_DIM), lambda i: (i, 0)),
        out_shape=jax.ShapeDtypeStruct((v, EMB_DIM), jnp.float32),
    )(wt)


@functools.partial(jax.jit, static_argnames=("n_chunks",))
def _embed(idx, table_p, n_chunks):
    n_per_w = n_chunks * CHUNK
    n_total = NUM_WORKERS * n_per_w

    mesh = plsc.VectorSubcoreMesh(
        core_axis_name="c", subcore_axis_name="s",
        num_cores=NUM_CORES, num_subcores=NUM_SUBCORES,
    )

    @functools.partial(
        pl.kernel,
        out_type=jax.ShapeDtypeStruct((n_total, EMB_DIM), jnp.float32),
        mesh=mesh,
        scratch_types=[
            pltpu.VMEM((n_chunks, CHUNK), jnp.int32),
            pltpu.VMEM((NBUF, CHUNK, EMB_DIM), jnp.float32),
            [pltpu.SemaphoreType.DMA] * NBUF,
            [pltpu.SemaphoreType.DMA] * NBUF,
        ],
    )
    def body(idx_hbm, table_hbm, out_hbm, idx_v, g_v, gsems, osems):
        wid = lax.axis_index("s") * NUM_CORES + lax.axis_index("c")
        base = wid * n_per_w
        pltpu.sync_copy(idx_hbm.at[wid], idx_v)

        def fire_gather(t, bb):
            def grp_dma(g, carry):
                t16 = idx_v[t, pl.ds(g * LANES, LANES)]
                for i in range(LANES):
                    pltpu.async_copy(
                        table_hbm.at[t16[i]],
                        g_v.at[bb, g * LANES + i], gsems[bb])
                return carry
            lax.fori_loop(0, CHUNK // LANES, grp_dma, 0)

        for u in range(NBUF - 1):
            fire_gather(u, u)

        def chunk_step(t, carry):
            b = lax.rem(t, NBUF)

            def per_buf(bb):
                pltpu.make_async_copy(
                    table_hbm.at[pl.ds(0, CHUNK)], g_v.at[bb],
                    gsems[bb]).wait()

                pltpu.async_copy(
                    g_v.at[bb],
                    out_hbm.at[pl.ds(base + t * CHUNK, CHUNK)], osems[bb])

                @pl.when(t + NBUF - 1 < n_chunks)
                def _():
                    bp = (bb - 1) % NBUF

                    @pl.when(t >= 1)
                    def _():
                        pltpu.make_async_copy(
                            g_v.at[bp],
                            out_hbm.at[
                                pl.ds(base + (t - 1) * CHUNK, CHUNK)],
                            osems[bp]).wait()
                    fire_gather(t + NBUF - 1, bp)

            for bb in range(NBUF):
                @pl.when(b == bb)
                def _(bb=bb):
                    per_buf(bb)
            return carry

        lax.fori_loop(0, n_chunks, chunk_step, 0)

        for k in range(NBUF):
            u = n_chunks - NBUF + k
            bu = u % NBUF
            pltpu.make_async_copy(
                g_v.at[bu],
                out_hbm.at[pl.ds(base + u * CHUNK, CHUNK)],
                osems[bu]).wait()

    return body(idx, table_p)


def kernel(tokens, embedding_weight):
    b, s = tokens.shape
    n = b * s
    assert n % (NUM_WORKERS * CHUNK) == 0
    n_chunks = n // (NUM_WORKERS * CHUNK)
    assert n_chunks >= NBUF
    idx = tokens.reshape(NUM_WORKERS, n_chunks, CHUNK).astype(jnp.int32)
    table_p = _transpose_scale(embedding_weight.T)
    out = _embed(idx, table_p, n_chunks)
    return out.reshape(b, s, EMB_DIM)


# speedup vs baseline: 1.1293x; 1.1293x over previous
import functools
import math

import jax
import jax.numpy as jnp
from jax import lax
from jax.experimental import pallas as pl
from jax.experimental.pallas import tpu as pltpu
from jax.experimental.pallas import tpu_sc as plsc

EMB_DIM = 64
SCALE = math.sqrt(EMB_DIM)

NUM_CORES = 2
NUM_SUBCORES = 16
NUM_WORKERS = NUM_CORES * NUM_SUBCORES
CHUNK = 128
LANES = 16
NBUF = 4


TBLK = 16384


def _transpose_scale(wt):
    v = wt.shape[1]

    def tbody(in_ref, out_ref):
        out_ref[...] = in_ref[...].T * SCALE

    return pl.pallas_call(
        tbody,
        grid=(pl.cdiv(v, TBLK),),
        in_specs=[pl.BlockSpec((EMB_DIM, TBLK), lambda i: (0, i))],
        out_specs=pl.BlockSpec((TBLK, EMB_DIM), lambda i: (i, 0)),
        out_shape=jax.ShapeDtypeStruct((v, EMB_DIM), jnp.float32),
    )(wt)


@functools.partial(jax.jit, static_argnames=("n_chunks",))
def _embed(idx, table_p, n_chunks):
    n_per_w = n_chunks * CHUNK
    n_total = NUM_WORKERS * n_per_w

    mesh = plsc.VectorSubcoreMesh(
        core_axis_name="c", subcore_axis_name="s",
        num_cores=NUM_CORES, num_subcores=NUM_SUBCORES,
    )

    @functools.partial(
        pl.kernel,
        out_type=jax.ShapeDtypeStruct((n_total, EMB_DIM), jnp.float32),
        mesh=mesh,
        scratch_types=[
            pltpu.VMEM((n_chunks, CHUNK), jnp.int32),
            pltpu.VMEM((NBUF, CHUNK, EMB_DIM), jnp.float32),
            [pltpu.SemaphoreType.DMA] * NBUF,
            [pltpu.SemaphoreType.DMA] * NBUF,
        ],
    )
    def body(idx_hbm, table_hbm, out_hbm, idx_v, g_v, gsems, osems):
        wid = lax.axis_index("s") * NUM_CORES + lax.axis_index("c")
        base = wid * n_per_w
        pltpu.sync_copy(idx_hbm.at[wid], idx_v)

        def fire_gather(t, bb):
            def grp_dma(g, carry):
                t16 = idx_v[t, pl.ds(g * LANES, LANES)]
                for i in range(LANES):
                    pltpu.async_copy(
                        table_hbm.at[t16[i]],
                        g_v.at[bb, g * LANES + i], gsems[bb])
                return carry
            lax.fori_loop(0, CHUNK // LANES, grp_dma, 0)

        for u in range(NBUF - 1):
            fire_gather(u, u)

        def chunk_step(t, carry):
            b = lax.rem(t, NBUF)

            def per_buf(bb):
                pltpu.make_async_copy(
                    table_hbm.at[pl.ds(0, CHUNK)], g_v.at[bb],
                    gsems[bb]).wait()

                pltpu.async_copy(
                    g_v.at[bb],
                    out_hbm.at[pl.ds(base + t * CHUNK, CHUNK)], osems[bb])

                @pl.when(t + NBUF - 1 < n_chunks)
                def _():
                    bp = (bb - 1) % NBUF

                    @pl.when(t >= 1)
                    def _():
                        pltpu.make_async_copy(
                            g_v.at[bp],
                            out_hbm.at[
                                pl.ds(base + (t - 1) * CHUNK, CHUNK)],
                            osems[bp]).wait()
                    fire_gather(t + NBUF - 1, bp)

            for bb in range(NBUF):
                @pl.when(b == bb)
                def _(bb=bb):
                    per_buf(bb)
            return carry

        lax.fori_loop(0, n_chunks, chunk_step, 0)

        for k in range(NBUF):
            u = n_chunks - NBUF + k
            bu = u % NBUF
            pltpu.make_async_copy(
                g_v.at[bu],
                out_hbm.at[pl.ds(base + u * CHUNK, CHUNK)],
                osems[bu]).wait()

    return body(idx, table_p)


def kernel(tokens, embedding_weight):
    b, s = tokens.shape
    n = b * s
    assert n % (NUM_WORKERS * CHUNK) == 0
    n_chunks = n // (NUM_WORKERS * CHUNK)
    assert n_chunks >= NBUF
    idx = tokens.reshape(NUM_WORKERS, n_chunks, CHUNK).astype(jnp.int32)
    table_p = _transpose_scale(embedding_weight.T)
    out = _embed(idx, table_p, n_chunks)
    return out.reshape(b, s, EMB_DIM)


# speedup vs baseline: 1.1390x; 1.0086x over previous
import functools
import math

import jax
import jax.numpy as jnp
from jax import lax
from jax.experimental import pallas as pl
from jax.experimental.pallas import tpu as pltpu
from jax.experimental.pallas import tpu_sc as plsc

EMB_DIM = 64
SCALE = math.sqrt(EMB_DIM)

NUM_CORES = 2
NUM_SUBCORES = 16
NUM_WORKERS = NUM_CORES * NUM_SUBCORES
CHUNK = 128
LANES = 16
NBUF = 4


TBLK = 32768


def _transpose_scale(wt):
    v = wt.shape[1]

    def tbody(in_ref, out_ref):
        out_ref[...] = in_ref[...].T * SCALE

    return pl.pallas_call(
        tbody,
        grid=(pl.cdiv(v, TBLK),),
        in_specs=[pl.BlockSpec((EMB_DIM, TBLK), lambda i: (0, i))],
        out_specs=pl.BlockSpec((TBLK, EMB_DIM), lambda i: (i, 0)),
        out_shape=jax.ShapeDtypeStruct((v, EMB_DIM), jnp.float32),
    )(wt)


@functools.partial(jax.jit, static_argnames=("n_chunks",))
def _embed(idx, table_p, n_chunks):
    n_per_w = n_chunks * CHUNK
    n_total = NUM_WORKERS * n_per_w

    mesh = plsc.VectorSubcoreMesh(
        core_axis_name="c", subcore_axis_name="s",
        num_cores=NUM_CORES, num_subcores=NUM_SUBCORES,
    )

    @functools.partial(
        pl.kernel,
        out_type=jax.ShapeDtypeStruct((n_total, EMB_DIM), jnp.float32),
        mesh=mesh,
        scratch_types=[
            pltpu.VMEM((n_chunks, CHUNK), jnp.int32),
            pltpu.VMEM((NBUF, CHUNK, EMB_DIM), jnp.float32),
            [pltpu.SemaphoreType.DMA] * NBUF,
            [pltpu.SemaphoreType.DMA] * NBUF,
        ],
    )
    def body(idx_hbm, table_hbm, out_hbm, idx_v, g_v, gsems, osems):
        wid = lax.axis_index("s") * NUM_CORES + lax.axis_index("c")
        base = wid * n_per_w
        pltpu.sync_copy(idx_hbm.at[wid], idx_v)

        def fire_gather(t, bb):
            def grp_dma(g, carry):
                t16 = idx_v[t, pl.ds(g * LANES, LANES)]
                for i in range(LANES):
                    pltpu.async_copy(
                        table_hbm.at[t16[i]],
                        g_v.at[bb, g * LANES + i], gsems[bb])
                return carry
            lax.fori_loop(0, CHUNK // LANES, grp_dma, 0)

        for u in range(NBUF - 1):
            fire_gather(u, u)

        def chunk_step(t, carry):
            b = lax.rem(t, NBUF)

            def per_buf(bb):
                pltpu.make_async_copy(
                    table_hbm.at[pl.ds(0, CHUNK)], g_v.at[bb],
                    gsems[bb]).wait()

                pltpu.async_copy(
                    g_v.at[bb],
                    out_hbm.at[pl.ds(base + t * CHUNK, CHUNK)], osems[bb])

                @pl.when(t + NBUF - 1 < n_chunks)
                def _():
                    bp = (bb - 1) % NBUF

                    @pl.when(t >= 1)
                    def _():
                        pltpu.make_async_copy(
                            g_v.at[bp],
                            out_hbm.at[
                                pl.ds(base + (t - 1) * CHUNK, CHUNK)],
                            osems[bp]).wait()
                    fire_gather(t + NBUF - 1, bp)

            for bb in range(NBUF):
                @pl.when(b == bb)
                def _(bb=bb):
                    per_buf(bb)
            return carry

        lax.fori_loop(0, n_chunks, chunk_step, 0)

        for k in range(NBUF):
            u = n_chunks - NBUF + k
            bu = u % NBUF
            pltpu.make_async_copy(
                g_v.at[bu],
                out_hbm.at[pl.ds(base + u * CHUNK, CHUNK)],
                osems[bu]).wait()

    return body(idx, table_p)


def kernel(tokens, embedding_weight):
    b, s = tokens.shape
    n = b * s
    assert n % (NUM_WORKERS * CHUNK) == 0
    n_chunks = n // (NUM_WORKERS * CHUNK)
    assert n_chunks >= NBUF
    idx = tokens.reshape(NUM_WORKERS, n_chunks, CHUNK).astype(jnp.int32)
    table_p = _transpose_scale(embedding_weight.T)
    out = _embed(idx, table_p, n_chunks)
    return out.reshape(b, s, EMB_DIM)
